# baseline (speedup 1.0000x reference)
import functools

import jax
import jax.numpy as jnp
from jax import lax
from jax.experimental import pallas as pl
from jax.experimental.pallas import tpu as pltpu
from jax.experimental.pallas import tpu_sc as plsc

NUM_DOMAINS = 4
EPS = 1e-3
B, D = 16384, 128
L = 16
NC, NS = 2, 16
COLS = D // NC
JG = COLS // L
RPT = B // NS
CH = 4
CR = RPT // CH
ACC_R = 16

_mesh = plsc.VectorSubcoreMesh(core_axis_name="c", subcore_axis_name="s")
_params = pltpu.CompilerParams(needs_layout_passes=False,
                               use_tc_tiling_on_sc=False)


def _iota16():
    return lax.broadcasted_iota(jnp.int32, (L,), 0)


def _rsqrt(x):
    i = plsc.bitcast(x, jnp.int32)
    y = plsc.bitcast(jnp.int32(0x5F3759DF) - (i >> 1), jnp.float32)
    for _ in range(3):
        y = y * (1.5 - 0.5 * x * y * y)
    return y


@functools.partial(
    pl.kernel,
    out_type=jax.ShapeDtypeStruct((B, D), jnp.float32),
    mesh=_mesh,
    compiler_params=_params,
    scratch_types=[
        pltpu.VMEM((CH, CR, COLS), jnp.float32),
        pltpu.VMEM((RPT * NUM_DOMAINS,), jnp.float32),
        pltpu.VMEM((RPT,), jnp.int32),
        pltpu.VMEM((ACC_R, COLS), jnp.float32),
        pltpu.VMEM((ACC_R, COLS), jnp.float32),
        pltpu.VMEM((2 * NUM_DOMAINS * COLS,), jnp.float32),
        pltpu.VMEM((NUM_DOMAINS, COLS), jnp.float32),
        pltpu.VMEM((NUM_DOMAINS, COLS), jnp.float32),
        pltpu.VMEM_SHARED((ACC_R, COLS), jnp.float32),
        pltpu.SemaphoreType.DMA,
        pltpu.SemaphoreType.DMA,
        pltpu.SemaphoreType.DMA,
        pltpu.SemaphoreType.DMA,
        pltpu.SemaphoreType.DMA,
        pltpu.SemaphoreType.DMA,
        pltpu.SemaphoreType.DMA,
        pltpu.SemaphoreType.DMA,
    ],
)
def _fused_kernel(x_hbm, di_hbm, g_hbm, b_hbm, out_hbm,
                  buf_v, di_v, didx_v, acc_v, tot_v, so_v, g_v, b_v, tot_sh,
                  i0, i1, i2, i3, o0, o1, o2, o3):
    cid = lax.axis_index("c")
    tid = lax.axis_index("s")
    col0 = cid * COLS
    row0 = tid * RPT
    iota = _iota16()
    isems = [i0, i1, i2, i3]
    osems = [o0, o1, o2, o3]

    def chunk_in(h):
        return pltpu.async_copy(
            x_hbm.at[pl.ds(row0 + h * CR, CR), pl.ds(col0, COLS)],
            buf_v.at[h],
            isems[h])

    cps = [chunk_in(h) for h in range(CH)]

    pltpu.sync_copy(di_hbm.at[pl.ds(row0 * NUM_DOMAINS, RPT * NUM_DOMAINS)],
                    di_v)
    pltpu.sync_copy(g_hbm.at[pl.ds(0, NUM_DOMAINS), pl.ds(col0, COLS)], g_v)
    pltpu.sync_copy(b_hbm.at[pl.ds(0, NUM_DOMAINS), pl.ds(col0, COLS)], b_v)

    zero = jnp.zeros((L,), jnp.float32)
    for q in range(ACC_R):
        for j in range(JG):
            acc_v[q, pl.ds(j * L, L)] = zero

    @pl.when(tid == 0)
    def _():
        pltpu.sync_copy(acc_v, tot_sh)

    plsc.subcore_barrier()

    def didx_body(t, _):
        idx0 = t * (L * NUM_DOMAINS) + iota * NUM_DOMAINS
        best = plsc.load_gather(di_v, [idx0])
        bidx = jnp.zeros((L,), jnp.int32)
        for c in range(1, NUM_DOMAINS):
            v = plsc.load_gather(di_v, [idx0 + c])
            take = v > best
            best = jnp.where(take, v, best)
            bidx = jnp.where(take, jnp.full((L,), c, jnp.int32), bidx)
        didx_v[pl.ds(t * L, L)] = bidx
        return 0

    lax.fori_loop(0, RPT // L, didx_body, 0, unroll=2)

    ones = jnp.ones((L,), jnp.float32)
    eights = jnp.full((L,), 8, jnp.int32)

    def make_acc_body(h):
        def acc_body(i, _):
            d_b = plsc.load_gather(
                didx_v, [jnp.zeros((L,), jnp.int32) + (i + h * CR)])
            vs = [buf_v[h, i, pl.ds(j * L, L)] for j in range(JG)]
            sqs = [v * v for v in vs]
            for j in range(JG):
                plsc.addupdate_scatter(acc_v, [d_b, j * L + iota], vs[j])
            for j in range(JG):
                plsc.addupdate_scatter(acc_v, [d_b + 4, j * L + iota], sqs[j])
            plsc.addupdate_scatter(acc_v, [eights, d_b * L + iota], ones)
            return 0
        return acc_body

    for h in range(CH):
        cps[h].wait()
        lax.fori_loop(0, CR, make_acc_body(h), 0, unroll=4)

    pltpu.sync_copy(acc_v, tot_sh.at[iota], add=True)
    plsc.subcore_barrier()
    pltpu.sync_copy(tot_sh, tot_v)

    for d in range(NUM_DOMAINS):
        cnt = tot_v[8, pl.ds(d * L, L)]
        safe = jnp.maximum(cnt, 1.0)
        for j in range(JG):
            off = d * COLS + j * L
            sm = tot_v[d, pl.ds(j * L, L)]
            sq = tot_v[d + 4, pl.ds(j * L, L)]
            mean = sm / safe
            var = jnp.maximum(sq / safe - mean * mean, 0.0)
            s = g_v[d, pl.ds(j * L, L)] * _rsqrt(var + EPS)
            so_v[pl.ds(off, L)] = s
            so_v[pl.ds(NUM_DOMAINS * COLS + off, L)] = \
                b_v[d, pl.ds(j * L, L)] - mean * s

    def make_norm_body(h):
        def norm_body(i, _):
            d_b = plsc.load_gather(
                didx_v, [jnp.zeros((L,), jnp.int32) + (i + h * CR)])
            col = d_b * COLS + iota
            vs = [buf_v[h, i, pl.ds(j * L, L)] for j in range(JG)]
            ss = [plsc.load_gather(so_v, [col + j * L]) for j in range(JG)]
            os_ = [plsc.load_gather(
                so_v, [col + j * L + NUM_DOMAINS * COLS]) for j in range(JG)]
            for j in range(JG):
                buf_v[h, i, pl.ds(j * L, L)] = vs[j] * ss[j] + os_[j]
            return 0
        return norm_body

    cps = [chunk_in(h) for h in range(CH)]
    ocps = []
    for h in range(CH):
        cps[h].wait()
        lax.fori_loop(0, CR, make_norm_body(h), 0, unroll=4)
        ocps.append(pltpu.async_copy(
            buf_v.at[h],
            out_hbm.at[pl.ds(row0 + h * CR, CR), pl.ds(col0, COLS)],
            osems[h]))
    for h in range(CH):
        ocps[h].wait()


def kernel(inputs, domain_indicator, gamma, beta):
    di = domain_indicator.reshape(-1)
    return _fused_kernel(inputs, di, gamma[:, :D], beta[:, :D])


# speedup vs baseline: 1.0425x; 1.0425x over previous
import functools

import jax
import jax.numpy as jnp
from jax import lax
from jax.experimental import pallas as pl
from jax.experimental.pallas import tpu as pltpu
from jax.experimental.pallas import tpu_sc as plsc

NUM_DOMAINS = 4
EPS = 1e-3
B, D = 16384, 128
L = 16
NC, NS = 2, 16
COLS = D // NC
JG = COLS // L
RPT = B // NS
CH = 8
CR = RPT // CH
ACC_R = 16

_mesh = plsc.VectorSubcoreMesh(core_axis_name="c", subcore_axis_name="s")
_params = pltpu.CompilerParams(needs_layout_passes=False,
                               use_tc_tiling_on_sc=False)


def _iota16():
    return lax.broadcasted_iota(jnp.int32, (L,), 0)


def _rsqrt(x):
    i = plsc.bitcast(x, jnp.int32)
    y = plsc.bitcast(jnp.int32(0x5F3759DF) - (i >> 1), jnp.float32)
    for _ in range(3):
        y = y * (1.5 - 0.5 * x * y * y)
    return y


@functools.partial(
    pl.kernel,
    out_type=jax.ShapeDtypeStruct((B, D), jnp.float32),
    mesh=_mesh,
    compiler_params=_params,
    scratch_types=[
        pltpu.VMEM((CH, CR, COLS), jnp.float32),
        pltpu.VMEM((RPT * NUM_DOMAINS,), jnp.float32),
        pltpu.VMEM((RPT,), jnp.int32),
        pltpu.VMEM((ACC_R, COLS), jnp.float32),
        pltpu.VMEM((ACC_R, COLS), jnp.float32),
        pltpu.VMEM((2 * NUM_DOMAINS * COLS,), jnp.float32),
        pltpu.VMEM((NUM_DOMAINS, COLS), jnp.float32),
        pltpu.VMEM((NUM_DOMAINS, COLS), jnp.float32),
        pltpu.VMEM_SHARED((ACC_R, COLS), jnp.float32),
    ] + [pltpu.SemaphoreType.DMA] * 16,
)
def _fused_kernel(x_hbm, di_hbm, g_hbm, b_hbm, out_hbm,
                  buf_v, di_v, didx_v, acc_v, tot_v, so_v, g_v, b_v, tot_sh,
                  *sems):
    cid = lax.axis_index("c")
    tid = lax.axis_index("s")
    col0 = cid * COLS
    row0 = tid * RPT
    iota = _iota16()
    isems = list(sems[:CH])
    osems = list(sems[CH:])

    def chunk_in(h):
        return pltpu.async_copy(
            x_hbm.at[pl.ds(row0 + h * CR, CR), pl.ds(col0, COLS)],
            buf_v.at[h],
            isems[h])

    cps = [chunk_in(h) for h in range(CH)]

    pltpu.sync_copy(di_hbm.at[pl.ds(row0 * NUM_DOMAINS, RPT * NUM_DOMAINS)],
                    di_v)
    pltpu.sync_copy(g_hbm.at[pl.ds(0, NUM_DOMAINS), pl.ds(col0, COLS)], g_v)
    pltpu.sync_copy(b_hbm.at[pl.ds(0, NUM_DOMAINS), pl.ds(col0, COLS)], b_v)

    zero = jnp.zeros((L,), jnp.float32)
    for q in range(ACC_R):
        for j in range(JG):
            acc_v[q, pl.ds(j * L, L)] = zero

    @pl.when(tid == 0)
    def _():
        pltpu.sync_copy(acc_v, tot_sh)

    plsc.subcore_barrier()

    ones = jnp.ones((L,), jnp.float32)
    eights = jnp.full((L,), 8, jnp.int32)

    def didx_body(t, _):
        idx0 = t * (L * NUM_DOMAINS) + iota * NUM_DOMAINS
        best = plsc.load_gather(di_v, [idx0])
        bidx = jnp.zeros((L,), jnp.int32)
        for c in range(1, NUM_DOMAINS):
            v = plsc.load_gather(di_v, [idx0 + c])
            take = v > best
            best = jnp.where(take, v, best)
            bidx = jnp.where(take, jnp.full((L,), c, jnp.int32), bidx)
        didx_v[pl.ds(t * L, L)] = bidx
        plsc.addupdate_scatter(acc_v, [eights, bidx * L + iota], ones)
        return 0

    lax.fori_loop(0, RPT // L, didx_body, 0, unroll=2)

    def make_acc_body(h):
        def acc_body(i, _):
            d_b = plsc.load_gather(
                didx_v, [jnp.zeros((L,), jnp.int32) + (i + h * CR)])
            vs = [buf_v[h, i, pl.ds(j * L, L)] for j in range(JG)]
            sqs = [v * v for v in vs]
            for j in range(JG):
                plsc.addupdate_scatter(acc_v, [d_b, j * L + iota], vs[j])
            for j in range(JG):
                plsc.addupdate_scatter(acc_v, [d_b + 4, j * L + iota], sqs[j])
            return 0
        return acc_body

    for h in range(CH):
        cps[h].wait()
        lax.fori_loop(0, CR, make_acc_body(h), 0, unroll=4)

    pltpu.sync_copy(acc_v, tot_sh.at[iota], add=True)
    plsc.subcore_barrier()
    pltpu.sync_copy(tot_sh, tot_v)

    for d in range(NUM_DOMAINS):
        cnt = jnp.sum(tot_v[8, pl.ds(d * L, L)])
        safe = jnp.maximum(cnt, 1.0)
        for j in range(JG):
            off = d * COLS + j * L
            sm = tot_v[d, pl.ds(j * L, L)]
            sq = tot_v[d + 4, pl.ds(j * L, L)]
            mean = sm / safe
            var = jnp.maximum(sq / safe - mean * mean, 0.0)
            s = g_v[d, pl.ds(j * L, L)] * _rsqrt(var + EPS)
            so_v[pl.ds(off, L)] = s
            so_v[pl.ds(NUM_DOMAINS * COLS + off, L)] = \
                b_v[d, pl.ds(j * L, L)] - mean * s

    def make_norm_body(h):
        def norm_body(i, _):
            d_b = plsc.load_gather(
                didx_v, [jnp.zeros((L,), jnp.int32) + (i + h * CR)])
            col = d_b * COLS + iota
            vs = [buf_v[h, i, pl.ds(j * L, L)] for j in range(JG)]
            ss = [plsc.load_gather(so_v, [col + j * L]) for j in range(JG)]
            os_ = [plsc.load_gather(
                so_v, [col + j * L + NUM_DOMAINS * COLS]) for j in range(JG)]
            for j in range(JG):
                buf_v[h, i, pl.ds(j * L, L)] = vs[j] * ss[j] + os_[j]
            return 0
        return norm_body

    ocps = []
    for h in range(CH):
        lax.fori_loop(0, CR, make_norm_body(h), 0, unroll=4)
        ocps.append(pltpu.async_copy(
            buf_v.at[h],
            out_hbm.at[pl.ds(row0 + h * CR, CR), pl.ds(col0, COLS)],
            osems[h]))
    for h in range(CH):
        ocps[h].wait()


def kernel(inputs, domain_indicator, gamma, beta):
    di = domain_indicator.reshape(-1)
    return _fused_kernel(inputs, di, gamma[:, :D], beta[:, :D])


# speedup vs baseline: 1.0493x; 1.0066x over previous
import functools

import jax
import jax.numpy as jnp
from jax import lax
from jax.experimental import pallas as pl
from jax.experimental.pallas import tpu as pltpu
from jax.experimental.pallas import tpu_sc as plsc

NUM_DOMAINS = 4
EPS = 1e-3
B, D = 16384, 128
L = 16
NC, NS = 2, 16
COLS = D // NC
JG = COLS // L
RPT = B // NS
CH = 8
CR = RPT // CH
ACC_R = 16

_mesh = plsc.VectorSubcoreMesh(core_axis_name="c", subcore_axis_name="s")
_params = pltpu.CompilerParams(needs_layout_passes=False,
                               use_tc_tiling_on_sc=False)


def _iota16():
    return lax.broadcasted_iota(jnp.int32, (L,), 0)


def _rsqrt(x):
    i = plsc.bitcast(x, jnp.int32)
    y = plsc.bitcast(jnp.int32(0x5F3759DF) - (i >> 1), jnp.float32)
    for _ in range(3):
        y = y * (1.5 - 0.5 * x * y * y)
    return y


@functools.partial(
    pl.kernel,
    out_type=jax.ShapeDtypeStruct((B, D), jnp.float32),
    mesh=_mesh,
    compiler_params=_params,
    scratch_types=[
        pltpu.VMEM((CH, CR, COLS), jnp.float32),
        pltpu.VMEM((RPT * NUM_DOMAINS,), jnp.float32),
        pltpu.VMEM((RPT,), jnp.int32),
        pltpu.VMEM((ACC_R, COLS), jnp.float32),
        pltpu.VMEM((ACC_R, COLS), jnp.float32),
        pltpu.VMEM((2 * NUM_DOMAINS * COLS,), jnp.float32),
        pltpu.VMEM((NUM_DOMAINS, COLS), jnp.float32),
        pltpu.VMEM((NUM_DOMAINS, COLS), jnp.float32),
        pltpu.VMEM_SHARED((ACC_R, COLS), jnp.float32),
    ] + [pltpu.SemaphoreType.DMA] * 16,
)
def _fused_kernel(x_hbm, di_hbm, g_hbm, b_hbm, out_hbm,
                  buf_v, di_v, didx_v, acc_v, tot_v, so_v, g_v, b_v, tot_sh,
                  *sems):
    cid = lax.axis_index("c")
    tid = lax.axis_index("s")
    col0 = cid * COLS
    row0 = tid * RPT
    iota = _iota16()
    isems = list(sems[:CH])
    osems = list(sems[CH:])

    def chunk_in(h):
        return pltpu.async_copy(
            x_hbm.at[pl.ds(row0 + h * CR, CR), pl.ds(col0, COLS)],
            buf_v.at[h],
            isems[h])

    pltpu.sync_copy(di_hbm.at[pl.ds(row0 * NUM_DOMAINS, RPT * NUM_DOMAINS)],
                    di_v)
    pltpu.sync_copy(g_hbm.at[pl.ds(0, NUM_DOMAINS), pl.ds(col0, COLS)], g_v)
    pltpu.sync_copy(b_hbm.at[pl.ds(0, NUM_DOMAINS), pl.ds(col0, COLS)], b_v)

    cps = [chunk_in(h) for h in range(CH)]

    zero = jnp.zeros((L,), jnp.float32)
    for q in range(ACC_R):
        for j in range(JG):
            acc_v[q, pl.ds(j * L, L)] = zero

    @pl.when(tid == 0)
    def _():
        pltpu.sync_copy(acc_v, tot_sh)

    plsc.subcore_barrier()

    ones = jnp.ones((L,), jnp.float32)
    eights = jnp.full((L,), 8, jnp.int32)

    def didx_body(t, _):
        idx0 = t * (L * NUM_DOMAINS) + iota * NUM_DOMAINS
        best = plsc.load_gather(di_v, [idx0])
        bidx = jnp.zeros((L,), jnp.int32)
        for c in range(1, NUM_DOMAINS):
            v = plsc.load_gather(di_v, [idx0 + c])
            take = v > best
            best = jnp.where(take, v, best)
            bidx = jnp.where(take, jnp.full((L,), c, jnp.int32), bidx)
        didx_v[pl.ds(t * L, L)] = bidx
        plsc.addupdate_scatter(acc_v, [eights, bidx * L + iota], ones)
        return 0

    lax.fori_loop(0, RPT // L, didx_body, 0, unroll=2)

    def make_acc_body(h):
        def acc_body(i, _):
            d_b = plsc.load_gather(
                didx_v, [jnp.zeros((L,), jnp.int32) + (i + h * CR)])
            vs = [buf_v[h, i, pl.ds(j * L, L)] for j in range(JG)]
            sqs = [v * v for v in vs]
            for j in range(JG):
                plsc.addupdate_scatter(acc_v, [d_b, j * L + iota], vs[j])
            for j in range(JG):
                plsc.addupdate_scatter(acc_v, [d_b + 4, j * L + iota], sqs[j])
            return 0
        return acc_body

    for h in range(CH):
        cps[h].wait()
        lax.fori_loop(0, CR, make_acc_body(h), 0, unroll=4)

    pltpu.sync_copy(acc_v, tot_sh.at[iota], add=True)
    plsc.subcore_barrier()
    pltpu.sync_copy(tot_sh, tot_v)

    for d in range(NUM_DOMAINS):
        cnt = jnp.sum(tot_v[8, pl.ds(d * L, L)])
        safe = jnp.maximum(cnt, 1.0)
        for j in range(JG):
            off = d * COLS + j * L
            sm = tot_v[d, pl.ds(j * L, L)]
            sq = tot_v[d + 4, pl.ds(j * L, L)]
            mean = sm / safe
            var = jnp.maximum(sq / safe - mean * mean, 0.0)
            s = g_v[d, pl.ds(j * L, L)] * _rsqrt(var + EPS)
            so_v[pl.ds(off, L)] = s
            so_v[pl.ds(NUM_DOMAINS * COLS + off, L)] = \
                b_v[d, pl.ds(j * L, L)] - mean * s

    def make_norm_body(h):
        def norm_body(i, _):
            d_b = plsc.load_gather(
                didx_v, [jnp.zeros((L,), jnp.int32) + (i + h * CR)])
            col = d_b * COLS + iota
            vs = [buf_v[h, i, pl.ds(j * L, L)] for j in range(JG)]
            ss = [plsc.load_gather(so_v, [col + j * L]) for j in range(JG)]
            os_ = [plsc.load_gather(
                so_v, [col + j * L + NUM_DOMAINS * COLS]) for j in range(JG)]
            for j in range(JG):
                buf_v[h, i, pl.ds(j * L, L)] = vs[j] * ss[j] + os_[j]
            return 0
        return norm_body

    ocps = []
    for h in range(CH):
        lax.fori_loop(0, CR, make_norm_body(h), 0, unroll=8)
        ocps.append(pltpu.async_copy(
            buf_v.at[h],
            out_hbm.at[pl.ds(row0 + h * CR, CR), pl.ds(col0, COLS)],
            osems[h]))
    for h in range(CH):
        ocps[h].wait()


def kernel(inputs, domain_indicator, gamma, beta):
    di = domain_indicator.reshape(-1)
    return _fused_kernel(inputs, di, gamma[:, :D], beta[:, :D])
